# baseline (speedup 1.0000x reference)
import jax
import jax.numpy as jnp
from jax import lax
from jax.experimental import pallas as pl
from jax.experimental.pallas import tpu as pltpu
from jax.experimental.pallas import tpu_sc as plsc

NODES_NUM = 1000
EMB_SIZE = 128
BATCH = 4096
SEQ = 50

NC, NS = 2, 16
NW = NC * NS
MOL_PER_W = BATCH // NW
OUT_W = EMB_SIZE + 2


def _sc_body(inp_hbm, table_hbm, out_hbm, inp_v, idx_v, rows_v, out_v, sem):
    wid = lax.axis_index("s") * NC + lax.axis_index("c")
    lanes = lax.iota(jnp.int32, 16)
    c0 = jnp.zeros((16,), jnp.int32)
    c1 = jnp.full((16,), 1, jnp.int32)
    c2 = jnp.full((16,), 2, jnp.int32)
    c129 = jnp.full((16,), OUT_W - 1, jnp.int32)

    @pl.loop(0, MOL_PER_W)
    def _mol(m):
        b = wid * MOL_PER_W + m
        pltpu.sync_copy(inp_hbm.at[b], inp_v)
        for i in range(4):
            rows = lanes + 16 * i
            rcl = jnp.minimum(rows, SEQ - 1)
            mask = rows < SEQ if i == 3 else None
            idx_v[0, pl.ds(i * 16, 16)] = plsc.load_gather(
                inp_v, [rcl, c0]).astype(jnp.int32)
            ch = plsc.load_gather(inp_v, [rcl, c1])
            rd = plsc.load_gather(inp_v, [rcl, c2])
            plsc.store_scatter(out_v, [rows, c0], ch, mask=mask)
            plsc.store_scatter(out_v, [rows, c129], rd, mask=mask)
        pltpu.async_copy(table_hbm.at[idx_v.at[0]], rows_v, sem).wait()

        @pl.loop(0, SEQ)
        def _row(r):
            for j in range(EMB_SIZE // 16):
                out_v[r, pl.ds(1 + j * 16, 16)] = rows_v[r, pl.ds(j * 16, 16)]

        pltpu.sync_copy(out_v, out_hbm.at[b])


@jax.jit
def kernel(inputs, emb_table):
    mesh = plsc.VectorSubcoreMesh(core_axis_name="c", subcore_axis_name="s")
    return pl.kernel(
        _sc_body,
        out_type=jax.ShapeDtypeStruct((BATCH, SEQ, OUT_W), jnp.float32),
        mesh=mesh,
        scratch_types=[
            pltpu.VMEM((SEQ, 3), jnp.float32),
            pltpu.VMEM((1, 64), jnp.int32),
            pltpu.VMEM((64, EMB_SIZE), jnp.float32),
            pltpu.VMEM((SEQ, OUT_W), jnp.float32),
            pltpu.SemaphoreType.DMA,
        ],
        compiler_params=pltpu.CompilerParams(
            use_tc_tiling_on_sc=False, needs_layout_passes=False),
    )(inputs, emb_table)
